# baseline (speedup 1.0000x reference)
import functools

import jax
import jax.numpy as jnp
from jax import lax
from jax.experimental import pallas as pl
from jax.experimental.pallas import tpu as pltpu
from jax.experimental.pallas import tpu_sc as plsc

NN = 10000
NP = 10240
EE = 320000
EP = 327680
H = 128
NG = 16
CW = 16
WIN = 128
BW = 512
BN = 512
NSC = 2
NSUB = 16
ROWS_SUB = NP // NSUB
WINS = EP // WIN
NWORK = NSC * NSUB
WPW = WINS // NWORK


def _silu(x):
    return x * jax.nn.sigmoid(x)


def _elu01(x):
    return jnp.where(x > 0, x, 0.1 * (jnp.exp(x) - 1.0))


def _mesh():
    return plsc.VectorSubcoreMesh(core_axis_name="c", subcore_axis_name="s")


_SC_PARAMS = pltpu.CompilerParams(use_tc_tiling_on_sc=False)


def _sc_gather(a_t, b_t, c_t, ri1, ci1):
    out_type = (jax.ShapeDtypeStruct((EP, H), jnp.bfloat16),
                jax.ShapeDtypeStruct((EP, H), jnp.bfloat16),
                jax.ShapeDtypeStruct((EP, CW), jnp.float32),
                jax.ShapeDtypeStruct((EP, CW), jnp.float32))
    scratch = [pltpu.VMEM((WPW * WIN,), jnp.int32),
               pltpu.VMEM((WPW * WIN,), jnp.int32),
               pltpu.VMEM((WIN, H), jnp.bfloat16),
               pltpu.VMEM((WIN, H), jnp.bfloat16),
               pltpu.VMEM((WIN, H), jnp.bfloat16),
               pltpu.VMEM((WIN, H), jnp.bfloat16),
               pltpu.VMEM((WIN, CW), jnp.float32),
               pltpu.VMEM((WIN, CW), jnp.float32),
               pltpu.VMEM((WIN, CW), jnp.float32),
               pltpu.VMEM((WIN, CW), jnp.float32),
               pltpu.SemaphoreType.DMA,
               pltpu.SemaphoreType.DMA]

    def body_fn(a_hbm, b_hbm, c_hbm, ri_hbm, ci_hbm,
                ga_hbm, gb_hbm, gcr_hbm, gcc_hbm,
                ri_a, ci_a, ga0, ga1, gb0, gb1, cr0, cr1, cc0, cc1, sem0, sem1):
        cid = lax.axis_index("c")
        sid = lax.axis_index("s")
        wid = sid * NSC + cid
        ebase = wid * WPW * WIN
        pltpu.sync_copy(ri_hbm.at[pl.ds(ebase, WPW * WIN)], ri_a)
        pltpu.sync_copy(ci_hbm.at[pl.ds(ebase, WPW * WIN)], ci_a)

        def fire(j, ga, gb, cr, cc, sem):
            rs = ri_a.at[pl.ds(j * WIN, WIN)]
            cs = ci_a.at[pl.ds(j * WIN, WIN)]
            pltpu.async_copy(a_hbm.at[rs], ga, sem)
            pltpu.async_copy(b_hbm.at[cs], gb, sem)
            pltpu.async_copy(c_hbm.at[rs], cr, sem)
            pltpu.async_copy(c_hbm.at[cs], cc, sem)

        def drain(ga, gb, cr, cc, sem):
            pltpu.make_async_copy(a_hbm.at[pl.ds(0, WIN)], ga, sem).wait()
            pltpu.make_async_copy(b_hbm.at[pl.ds(0, WIN)], gb, sem).wait()
            pltpu.make_async_copy(c_hbm.at[pl.ds(0, WIN)], cr, sem).wait()
            pltpu.make_async_copy(c_hbm.at[pl.ds(0, WIN)], cc, sem).wait()

        def store(j, ga, gb, cr, cc):
            base = ebase + j * WIN
            pltpu.sync_copy(ga, ga_hbm.at[pl.ds(base, WIN)])
            pltpu.sync_copy(gb, gb_hbm.at[pl.ds(base, WIN)])
            pltpu.sync_copy(cr, gcr_hbm.at[pl.ds(base, WIN)])
            pltpu.sync_copy(cc, gcc_hbm.at[pl.ds(base, WIN)])

        fire(0, ga0, gb0, cr0, cc0, sem0)

        @pl.loop(0, WPW // 2)
        def _(k):
            j0 = 2 * k
            drain(ga0, gb0, cr0, cc0, sem0)
            fire(j0 + 1, ga1, gb1, cr1, cc1, sem1)
            store(j0, ga0, gb0, cr0, cc0)
            drain(ga1, gb1, cr1, cc1, sem1)

            @pl.when(k < WPW // 2 - 1)
            def _():
                fire(j0 + 2, ga0, gb0, cr0, cc0, sem0)

            store(j0 + 1, ga1, gb1, cr1, cc1)

    k = pl.kernel(body_fn, out_type=out_type, mesh=_mesh(), scratch_types=scratch,
                  compiler_params=_SC_PARAMS)
    return k(a_t, b_t, c_t, ri1, ci1)


def _sc_scatter(ef, cm, ri1, zh, zc):
    out_type = (jax.ShapeDtypeStruct((NSC, NP, H), jnp.float32),
                jax.ShapeDtypeStruct((NSC, NP, CW), jnp.float32))
    scratch = [pltpu.VMEM((WIN,), jnp.int32),
               pltpu.VMEM((WIN, H), jnp.float32),
               pltpu.VMEM((WIN, CW), jnp.float32),
               pltpu.VMEM_SHARED((NP, H), jnp.float32),
               pltpu.VMEM_SHARED((NP, CW), jnp.float32)]

    def body_fn(ef_hbm, cm_hbm, ri_hbm, zh_hbm, zc_hbm, oa_hbm, oc_hbm,
                idx_v, efb_v, cmb_v, acc_s, cacc_s):
        cid = lax.axis_index("c")
        sid = lax.axis_index("s")
        r0 = sid * ROWS_SUB
        pltpu.sync_copy(zh_hbm.at[pl.ds(r0, ROWS_SUB)], acc_s.at[pl.ds(r0, ROWS_SUB)])
        pltpu.sync_copy(zc_hbm.at[pl.ds(r0, ROWS_SUB)], cacc_s.at[pl.ds(r0, ROWS_SUB)])
        plsc.subcore_barrier()
        wid = sid * NSC + cid

        @pl.loop(0, WPW)
        def _(j):
            base = (wid * WPW + j) * WIN
            pltpu.sync_copy(ri_hbm.at[pl.ds(base, WIN)], idx_v)
            pltpu.sync_copy(ef_hbm.at[pl.ds(base, WIN)], efb_v)
            pltpu.sync_copy(cm_hbm.at[pl.ds(base, WIN)], cmb_v)
            pltpu.sync_copy(efb_v, acc_s.at[idx_v], add=True)
            pltpu.sync_copy(cmb_v, cacc_s.at[idx_v], add=True)

        plsc.subcore_barrier()
        pltpu.sync_copy(acc_s.at[pl.ds(r0, ROWS_SUB)], oa_hbm.at[cid, pl.ds(r0, ROWS_SUB)])
        pltpu.sync_copy(cacc_s.at[pl.ds(r0, ROWS_SUB)], oc_hbm.at[cid, pl.ds(r0, ROWS_SUB)])

    k = pl.kernel(body_fn, out_type=out_type, mesh=_mesh(), scratch_types=scratch, compiler_params=_SC_PARAMS)
    return k(ef, cm, ri1, zh, zc)


def _sc_gin(xw, ri1, ci1, zh):
    out_type = jax.ShapeDtypeStruct((NSC, NP, H), jnp.float32)
    scratch = [pltpu.VMEM((WIN,), jnp.int32),
               pltpu.VMEM((WIN,), jnp.int32),
               pltpu.VMEM((WIN, H), jnp.float32),
               pltpu.VMEM_SHARED((NP, H), jnp.float32)]

    def body_fn(xw_hbm, ri_hbm, ci_hbm, zh_hbm, oa_hbm, ri_v, ci_v, buf_v, acc_s):
        cid = lax.axis_index("c")
        sid = lax.axis_index("s")
        r0 = sid * ROWS_SUB
        pltpu.sync_copy(zh_hbm.at[pl.ds(r0, ROWS_SUB)], acc_s.at[pl.ds(r0, ROWS_SUB)])
        plsc.subcore_barrier()
        wid = sid * NSC + cid

        @pl.loop(0, WPW)
        def _(j):
            base = (wid * WPW + j) * WIN
            pltpu.sync_copy(ri_hbm.at[pl.ds(base, WIN)], ri_v)
            pltpu.sync_copy(ci_hbm.at[pl.ds(base, WIN)], ci_v)
            pltpu.sync_copy(xw_hbm.at[ri_v], buf_v)
            pltpu.sync_copy(buf_v, acc_s.at[ci_v], add=True)

        plsc.subcore_barrier()
        pltpu.sync_copy(acc_s.at[pl.ds(r0, ROWS_SUB)], oa_hbm.at[cid, pl.ds(r0, ROWS_SUB)])

    k = pl.kernel(body_fn, out_type=out_type, mesh=_mesh(), scratch_types=scratch, compiler_params=_SC_PARAMS)
    return k(xw, ri1, ci1, zh)


def _full(shape):
    nd = len(shape)
    return pl.BlockSpec(shape, lambda i: (0,) * nd)


def _tc_prelude(hp, w1a, w1b):
    def kfn(h_ref, wa_ref, wb_ref, a_ref, b_ref):
        hb = h_ref[...]
        a_ref[...] = jnp.dot(hb, wa_ref[...], preferred_element_type=jnp.float32).astype(jnp.bfloat16)
        b_ref[...] = jnp.dot(hb, wb_ref[...], preferred_element_type=jnp.float32).astype(jnp.bfloat16)

    return pl.pallas_call(
        kfn,
        grid=(NP // BN,),
        in_specs=[pl.BlockSpec((BN, H), lambda i: (i, 0)), _full((H, H)), _full((H, H))],
        out_specs=[pl.BlockSpec((BN, H), lambda i: (i, 0)),
                   pl.BlockSpec((BN, H), lambda i: (i, 0))],
        out_shape=(jax.ShapeDtypeStruct((NP, H), jnp.bfloat16),
                   jax.ShapeDtypeStruct((NP, H), jnp.bfloat16)),
    )(hp, w1a, w1b)


def _tc_edge(ga, gb, gcr, gcc, w1r, eb1, ew2, eb2, cw1, cb1, cw2t):
    def kfn(ga_ref, gb_ref, gcr_ref, gcc_ref, w1r_ref, eb1_ref, ew2_ref, eb2_ref,
            cw1_ref, cb1_ref, cw2t_ref, ef_ref, cm_ref):
        cd = gcr_ref[...] - gcc_ref[...]
        radial = jnp.sum(cd * cd, axis=1, keepdims=True)
        pre = (ga_ref[...].astype(jnp.float32) + gb_ref[...].astype(jnp.float32)
               + radial * w1r_ref[...] + eb1_ref[...])
        ef1 = _silu(pre)
        ef2 = _silu(jnp.dot(ef1, ew2_ref[...], preferred_element_type=jnp.float32) + eb2_ref[...])
        u = _silu(jnp.dot(ef2, cw1_ref[...], preferred_element_type=jnp.float32) + cb1_ref[...])
        m = jnp.sum(u * cw2t_ref[...], axis=1, keepdims=True)
        lane = lax.broadcasted_iota(jnp.int32, (BW, CW), 1)
        cm = jnp.where(lane == 3, 1.0, cd * m)
        ef_ref[...] = ef2
        cm_ref[...] = cm

    return pl.pallas_call(
        kfn,
        grid=(EP // BW,),
        in_specs=[pl.BlockSpec((BW, H), lambda i: (i, 0)),
                  pl.BlockSpec((BW, H), lambda i: (i, 0)),
                  pl.BlockSpec((BW, CW), lambda i: (i, 0)),
                  pl.BlockSpec((BW, CW), lambda i: (i, 0)),
                  _full((1, H)), _full((1, H)), _full((H, H)), _full((1, H)),
                  _full((H, H)), _full((1, H)), _full((1, H))],
        out_specs=[pl.BlockSpec((BW, H), lambda i: (i, 0)),
                   pl.BlockSpec((BW, CW), lambda i: (i, 0))],
        out_shape=(jax.ShapeDtypeStruct((EP, H), jnp.float32),
                   jax.ShapeDtypeStruct((EP, CW), jnp.float32)),
    )(ga, gb, gcr, gcc, w1r, eb1, ew2, eb2, cw1, cb1, cw2t)


def _tc_node0(hp, aggp, caccp, coordp, nw1a, nw1b, nb1, nw2, nb2, w1a_n, w1b_n):
    def kfn(h_ref, ap_ref, cp_ref, co_ref, nw1a_ref, nw1b_ref, nb1_ref, nw2_ref,
            nb2_ref, wa_ref, wb_ref, hn_ref, con_ref, an_ref, bn_ref):
        agg = ap_ref[0] + ap_ref[1]
        o1 = _silu(jnp.dot(h_ref[...], nw1a_ref[...], preferred_element_type=jnp.float32)
                   + jnp.dot(agg, nw1b_ref[...], preferred_element_type=jnp.float32)
                   + nb1_ref[...])
        hn = jnp.dot(o1, nw2_ref[...], preferred_element_type=jnp.float32) + nb2_ref[...]
        cacc = cp_ref[0] + cp_ref[1]
        cnt = jnp.clip(cacc[:, 3:4], 1.0, None)
        lane = lax.broadcasted_iota(jnp.int32, (BN, CW), 1)
        con = co_ref[...] + jnp.where(lane < 3, cacc / cnt, 0.0)
        hn_ref[...] = hn
        con_ref[...] = con
        an_ref[...] = jnp.dot(hn, wa_ref[...], preferred_element_type=jnp.float32).astype(jnp.bfloat16)
        bn_ref[...] = jnp.dot(hn, wb_ref[...], preferred_element_type=jnp.float32).astype(jnp.bfloat16)

    return pl.pallas_call(
        kfn,
        grid=(NP // BN,),
        in_specs=[pl.BlockSpec((BN, H), lambda i: (i, 0)),
                  pl.BlockSpec((NSC, BN, H), lambda i: (0, i, 0)),
                  pl.BlockSpec((NSC, BN, CW), lambda i: (0, i, 0)),
                  pl.BlockSpec((BN, CW), lambda i: (i, 0)),
                  _full((H, H)), _full((H, H)), _full((1, H)), _full((H, H)),
                  _full((1, H)), _full((H, H)), _full((H, H))],
        out_specs=[pl.BlockSpec((BN, H), lambda i: (i, 0)),
                   pl.BlockSpec((BN, CW), lambda i: (i, 0)),
                   pl.BlockSpec((BN, H), lambda i: (i, 0)),
                   pl.BlockSpec((BN, H), lambda i: (i, 0))],
        out_shape=(jax.ShapeDtypeStruct((NP, H), jnp.float32),
                   jax.ShapeDtypeStruct((NP, CW), jnp.float32),
                   jax.ShapeDtypeStruct((NP, H), jnp.bfloat16),
                   jax.ShapeDtypeStruct((NP, H), jnp.bfloat16)),
    )(hp, aggp, caccp, coordp, nw1a, nw1b, nb1, nw2, nb2, w1a_n, w1b_n)


def _tc_node1(hp, aggp, caccp, coordp, coord0p, nw1a, nw1b, nb1, nw2, nb2, gw1a, gw1b):
    def kfn(h_ref, ap_ref, cp_ref, co_ref, co0_ref, nw1a_ref, nw1b_ref, nb1_ref,
            nw2_ref, nb2_ref, ga_ref, gb_ref, xw_ref):
        agg = ap_ref[0] + ap_ref[1]
        o1 = _silu(jnp.dot(h_ref[...], nw1a_ref[...], preferred_element_type=jnp.float32)
                   + jnp.dot(agg, nw1b_ref[...], preferred_element_type=jnp.float32)
                   + nb1_ref[...])
        hn = h_ref[...] + jnp.dot(o1, nw2_ref[...], preferred_element_type=jnp.float32) + nb2_ref[...]
        cacc = cp_ref[0] + cp_ref[1]
        cnt = jnp.clip(cacc[:, 3:4], 1.0, None)
        lane = lax.broadcasted_iota(jnp.int32, (BN, CW), 1)
        con = co_ref[...] + jnp.where(lane < 3, cacc / cnt, 0.0)
        dd = con - co0_ref[...]
        delta = jnp.sqrt(jnp.sum(dd * dd, axis=1, keepdims=True))
        xw_ref[...] = (jnp.dot(hn, ga_ref[...], preferred_element_type=jnp.float32)
                       + delta * gb_ref[...])

    return pl.pallas_call(
        kfn,
        grid=(NP // BN,),
        in_specs=[pl.BlockSpec((BN, H), lambda i: (i, 0)),
                  pl.BlockSpec((NSC, BN, H), lambda i: (0, i, 0)),
                  pl.BlockSpec((NSC, BN, CW), lambda i: (0, i, 0)),
                  pl.BlockSpec((BN, CW), lambda i: (i, 0)),
                  pl.BlockSpec((BN, CW), lambda i: (i, 0)),
                  _full((H, H)), _full((H, H)), _full((1, H)), _full((H, H)),
                  _full((1, H)), _full((H, H)), _full((1, H))],
        out_specs=[pl.BlockSpec((BN, H), lambda i: (i, 0))],
        out_shape=(jax.ShapeDtypeStruct((NP, H), jnp.float32),),
    )(hp, aggp, caccp, coordp, coord0p, nw1a, nw1b, nb1, nw2, nb2, gw1a, gw1b)


def _tc_final(xw, gp, batc, g_b1, g_gamma, g_beta, g_w2, g_b2, g0,
              cw1m, cw1x, cw1g, c_b1, c_gamma, c_beta, c_w2, c_b2):
    nblk = NP // BN
    binv = float((1.0 + 1e-5) ** -0.5)

    def kfn(xw_ref, gp_ref, bat_ref, gb1_ref, gg_ref, gbe_ref, gw2_ref, gb2_ref,
            g0_ref, c1m_ref, c1x_ref, c1g_ref, cb1_ref, cg_ref, cbe_ref,
            cw2_ref, cb2_ref, out_ref, mean_s, max_s, cnt_s):
        step = pl.program_id(0)

        @pl.when(step == 0)
        def _():
            mean_s[...] = jnp.zeros((NG, H), jnp.float32)
            cnt_s[...] = jnp.zeros((NG, H), jnp.float32)
            max_s[...] = jnp.full((NG, H), -jnp.inf, jnp.float32)

        y = xw_ref[...] + gp_ref[0] + gp_ref[1] + gb1_ref[...]
        y = y * binv * gg_ref[...] + gbe_ref[...]
        y = jnp.maximum(y, 0.0)
        hf = _elu01(jnp.dot(y, gw2_ref[...], preferred_element_type=jnp.float32) + gb2_ref[...])

        bat = bat_ref[...]
        grp = lax.broadcasted_iota(jnp.int32, (NG, BN), 0)
        pmat = (bat.reshape(1, BN) == grp).astype(jnp.float32)
        mean_s[...] += jnp.dot(pmat, hf, preferred_element_type=jnp.float32)
        cnt_s[...] += jnp.broadcast_to(jnp.sum(pmat, axis=1, keepdims=True), (NG, H))
        for g in range(NG):
            mg = jnp.where(bat == g, hf, -jnp.inf)
            red = jnp.max(mg, axis=0, keepdims=True)
            max_s[g:g + 1, :] = jnp.maximum(max_s[g:g + 1, :], red)

        @pl.when(step == nblk - 1)
        def _():
            cnt = jnp.clip(cnt_s[...], 1.0, None)
            meanp = mean_s[...] / cnt
            maxp = max_s[...]
            z = (jnp.dot(meanp, c1m_ref[...], preferred_element_type=jnp.float32)
                 + jnp.dot(maxp, c1x_ref[...], preferred_element_type=jnp.float32)
                 + jnp.dot(g0_ref[...], c1g_ref[...], preferred_element_type=jnp.float32)
                 + cb1_ref[...])
            z = _elu01(z)
            z = z * binv * cg_ref[...] + cbe_ref[...]
            z = jnp.dot(z, cw2_ref[...], preferred_element_type=jnp.float32) + cb2_ref[...]
            z = z - jnp.max(z, axis=1, keepdims=True)
            ez = jnp.exp(z)
            out_ref[...] = ez / jnp.sum(ez, axis=1, keepdims=True)

    return pl.pallas_call(
        kfn,
        grid=(nblk,),
        in_specs=[pl.BlockSpec((BN, H), lambda i: (i, 0)),
                  pl.BlockSpec((NSC, BN, H), lambda i: (0, i, 0)),
                  pl.BlockSpec((BN, 1), lambda i: (i, 0)),
                  _full((1, H)), _full((1, H)), _full((1, H)), _full((H, H)),
                  _full((1, H)), _full((NG, NG)), _full((H, H)), _full((H, H)),
                  _full((NG, H)), _full((1, H)), _full((1, H)), _full((1, H)),
                  _full((H, 4)), _full((1, 4))],
        out_specs=pl.BlockSpec((NG, 4), lambda i: (0, 0)),
        out_shape=jax.ShapeDtypeStruct((NG, 4), jnp.float32),
        scratch_shapes=[pltpu.VMEM((NG, H), jnp.float32),
                        pltpu.VMEM((NG, H), jnp.float32),
                        pltpu.VMEM((NG, H), jnp.float32)],
    )(xw, gp, batc, g_b1, g_gamma, g_beta, g_w2, g_b2, g0,
      cw1m, cw1x, cw1g, c_b1, c_gamma, c_beta, c_w2, c_b2)


def kernel(h0, coord0, g0, edge_index, batch, e0_ew1, e0_eb1, e0_ew2, e0_eb2, e0_nw1, e0_nb1, e0_nw2, e0_nb2, e0_cw1, e0_cb1, e0_cw2, e1_ew1, e1_eb1, e1_ew2, e1_eb2, e1_nw1, e1_nb1, e1_nw2, e1_nb2, e1_cw1, e1_cb1, e1_cw2, g_w1, g_b1, g_gamma, g_beta, g_w2, g_b2, c_w1, c_b1, c_gamma, c_beta, c_w2, c_b2):
    f32 = jnp.float32
    hp = jnp.zeros((NP, H), f32).at[:NN].set(h0)
    coord0p = jnp.zeros((NP, CW), f32).at[:NN, :3].set(coord0)
    row = jnp.full((EP,), NN, jnp.int32).at[:EE].set(edge_index[0])
    col = jnp.full((EP,), NN, jnp.int32).at[:EE].set(edge_index[1])
    batc = jnp.full((NP, 1), NG, jnp.int32).at[:NN, 0].set(batch)
    zh = jnp.zeros((NP, H), f32)
    zc = jnp.zeros((NP, CW), f32)

    def rowvec(v):
        return v.reshape(1, -1)

    coordp = coord0p
    e_params = [
        (e0_ew1, e0_eb1, e0_ew2, e0_eb2, e0_nw1, e0_nb1, e0_nw2, e0_nb2, e0_cw1, e0_cb1, e0_cw2),
        (e1_ew1, e1_eb1, e1_ew2, e1_eb2, e1_nw1, e1_nb1, e1_nw2, e1_nb2, e1_cw1, e1_cb1, e1_cw2),
    ]

    (ew1, eb1, ew2, eb2, nw1, nb1, nw2, nb2, cw1, cb1, cw2) = e_params[0]
    a_t, b_t = _tc_prelude(hp, ew1[0:H], ew1[H:2 * H])
    ga, gb, gcr, gcc = _sc_gather(a_t, b_t, coordp, row, col)
    ef, cm = _tc_edge(ga, gb, gcr, gcc, rowvec(ew1[2 * H]), rowvec(eb1),
                      ew2, rowvec(eb2), cw1, rowvec(cb1), cw2.reshape(1, H))
    aggp, caccp = _sc_scatter(ef, cm, row, zh, zc)
    ew1n = e_params[1][0]
    hp, coordp, a_t, b_t = _tc_node0(hp, aggp, caccp, coordp,
                                     nw1[0:H], nw1[H:2 * H], rowvec(nb1),
                                     nw2, rowvec(nb2), ew1n[0:H], ew1n[H:2 * H])

    (ew1, eb1, ew2, eb2, nw1, nb1, nw2, nb2, cw1, cb1, cw2) = e_params[1]
    ga, gb, gcr, gcc = _sc_gather(a_t, b_t, coordp, row, col)
    ef, cm = _tc_edge(ga, gb, gcr, gcc, rowvec(ew1[2 * H]), rowvec(eb1),
                      ew2, rowvec(eb2), cw1, rowvec(cb1), cw2.reshape(1, H))
    aggp, caccp = _sc_scatter(ef, cm, row, zh, zc)
    xw = _tc_node1(hp, aggp, caccp, coordp, coord0p,
                   nw1[0:H], nw1[H:2 * H], rowvec(nb1), nw2, rowvec(nb2),
                   g_w1[0:H], rowvec(g_w1[H]))[0]

    gp = _sc_gin(xw, row, col, zh)

    return _tc_final(xw, gp, batc, rowvec(g_b1), rowvec(g_gamma), rowvec(g_beta),
                     g_w2, rowvec(g_b2), g0, c_w1[0:H], c_w1[H:2 * H],
                     c_w1[2 * H:2 * H + NG], rowvec(c_b1), rowvec(c_gamma),
                     rowvec(c_beta), c_w2, rowvec(c_b2))


# speedup vs baseline: 1.3918x; 1.3918x over previous
import functools

import jax
import jax.numpy as jnp
from jax import lax
from jax.experimental import pallas as pl
from jax.experimental.pallas import tpu as pltpu
from jax.experimental.pallas import tpu_sc as plsc

NN = 10000
NP = 10240
EE = 320000
EP = 327680
H = 128
NG = 16
CW = 16
WIN = 128
BW = 512
BN = 512
NSC = 2
NSUB = 16
ROWS_SUB = NP // NSUB
WINS = EP // WIN
NWORK = NSC * NSUB
WPW = WINS // NWORK


def _silu(x):
    return x * jax.nn.sigmoid(x)


def _elu01(x):
    return jnp.where(x > 0, x, 0.1 * (jnp.exp(x) - 1.0))


def _mesh():
    return plsc.VectorSubcoreMesh(core_axis_name="c", subcore_axis_name="s")


_SC_PARAMS = pltpu.CompilerParams(use_tc_tiling_on_sc=False)


def _sc_gather(a_t, b_t, c_t, ri1, ci1, ne):
    wpw = ne // WIN // NWORK
    out_type = (jax.ShapeDtypeStruct((ne, H), jnp.float32),
                jax.ShapeDtypeStruct((ne, H), jnp.float32),
                jax.ShapeDtypeStruct((ne, CW), jnp.float32),
                jax.ShapeDtypeStruct((ne, CW), jnp.float32))
    scratch = [pltpu.VMEM((wpw * WIN,), jnp.int32),
               pltpu.VMEM((wpw * WIN,), jnp.int32),
               pltpu.VMEM((WIN, H), jnp.float32),
               pltpu.VMEM((WIN, H), jnp.float32),
               pltpu.VMEM((WIN, H), jnp.float32),
               pltpu.VMEM((WIN, H), jnp.float32),
               pltpu.VMEM((WIN, CW), jnp.float32),
               pltpu.VMEM((WIN, CW), jnp.float32),
               pltpu.VMEM((WIN, CW), jnp.float32),
               pltpu.VMEM((WIN, CW), jnp.float32),
               pltpu.SemaphoreType.DMA,
               pltpu.SemaphoreType.DMA]

    def body_fn(a_hbm, b_hbm, c_hbm, ri_hbm, ci_hbm,
                ga_hbm, gb_hbm, gcr_hbm, gcc_hbm,
                ri_a, ci_a, ga0, ga1, gb0, gb1, cr0, cr1, cc0, cc1, sem0, sem1):
        cid = lax.axis_index("c")
        sid = lax.axis_index("s")
        wid = sid * NSC + cid
        ebase = wid * wpw * WIN
        pltpu.sync_copy(ri_hbm.at[pl.ds(ebase, wpw * WIN)], ri_a)
        pltpu.sync_copy(ci_hbm.at[pl.ds(ebase, wpw * WIN)], ci_a)

        def fire(j, ga, gb, cr, cc, sem):
            rs = ri_a.at[pl.ds(j * WIN, WIN)]
            cs = ci_a.at[pl.ds(j * WIN, WIN)]
            pltpu.async_copy(a_hbm.at[rs], ga, sem)
            pltpu.async_copy(b_hbm.at[cs], gb, sem)
            pltpu.async_copy(c_hbm.at[rs], cr, sem)
            pltpu.async_copy(c_hbm.at[cs], cc, sem)

        def drain(ga, gb, cr, cc, sem):
            pltpu.make_async_copy(a_hbm.at[pl.ds(0, WIN)], ga, sem).wait()
            pltpu.make_async_copy(b_hbm.at[pl.ds(0, WIN)], gb, sem).wait()
            pltpu.make_async_copy(c_hbm.at[pl.ds(0, WIN)], cr, sem).wait()
            pltpu.make_async_copy(c_hbm.at[pl.ds(0, WIN)], cc, sem).wait()

        def store(j, ga, gb, cr, cc):
            base = ebase + j * WIN
            pltpu.sync_copy(ga, ga_hbm.at[pl.ds(base, WIN)])
            pltpu.sync_copy(gb, gb_hbm.at[pl.ds(base, WIN)])
            pltpu.sync_copy(cr, gcr_hbm.at[pl.ds(base, WIN)])
            pltpu.sync_copy(cc, gcc_hbm.at[pl.ds(base, WIN)])

        fire(0, ga0, gb0, cr0, cc0, sem0)

        @pl.loop(0, wpw // 2)
        def _(k):
            j0 = 2 * k
            drain(ga0, gb0, cr0, cc0, sem0)
            fire(j0 + 1, ga1, gb1, cr1, cc1, sem1)
            store(j0, ga0, gb0, cr0, cc0)
            drain(ga1, gb1, cr1, cc1, sem1)

            @pl.when(k < wpw // 2 - 1)
            def _():
                fire(j0 + 2, ga0, gb0, cr0, cc0, sem0)

            store(j0 + 1, ga1, gb1, cr1, cc1)

    k = pl.kernel(body_fn, out_type=out_type, mesh=_mesh(), scratch_types=scratch,
                  compiler_params=_SC_PARAMS)
    return k(a_t, b_t, c_t, ri1, ci1)


def _sc_scatter(ef, cm, ri1, zh, zc, ne):
    wpw = ne // WIN // NWORK
    out_type = (jax.ShapeDtypeStruct((NSC, NP, H), jnp.float32),
                jax.ShapeDtypeStruct((NSC, NP, CW), jnp.float32))
    scratch = [pltpu.VMEM((WIN,), jnp.int32),
               pltpu.VMEM((WIN, H), jnp.float32),
               pltpu.VMEM((WIN, CW), jnp.float32),
               pltpu.VMEM_SHARED((NP, H), jnp.float32),
               pltpu.VMEM_SHARED((NP, CW), jnp.float32)]

    def body_fn(ef_hbm, cm_hbm, ri_hbm, zh_hbm, zc_hbm, oa_hbm, oc_hbm,
                idx_v, efb_v, cmb_v, acc_s, cacc_s):
        cid = lax.axis_index("c")
        sid = lax.axis_index("s")
        r0 = sid * ROWS_SUB
        pltpu.sync_copy(zh_hbm.at[pl.ds(r0, ROWS_SUB)], acc_s.at[pl.ds(r0, ROWS_SUB)])
        pltpu.sync_copy(zc_hbm.at[pl.ds(r0, ROWS_SUB)], cacc_s.at[pl.ds(r0, ROWS_SUB)])
        plsc.subcore_barrier()
        wid = sid * NSC + cid

        @pl.loop(0, wpw)
        def _(j):
            base = (wid * wpw + j) * WIN
            pltpu.sync_copy(ri_hbm.at[pl.ds(base, WIN)], idx_v)
            pltpu.sync_copy(ef_hbm.at[pl.ds(base, WIN)], efb_v)
            pltpu.sync_copy(cm_hbm.at[pl.ds(base, WIN)], cmb_v)
            pltpu.sync_copy(efb_v, acc_s.at[idx_v], add=True)
            pltpu.sync_copy(cmb_v, cacc_s.at[idx_v], add=True)

        plsc.subcore_barrier()
        pltpu.sync_copy(acc_s.at[pl.ds(r0, ROWS_SUB)], oa_hbm.at[cid, pl.ds(r0, ROWS_SUB)])
        pltpu.sync_copy(cacc_s.at[pl.ds(r0, ROWS_SUB)], oc_hbm.at[cid, pl.ds(r0, ROWS_SUB)])

    k = pl.kernel(body_fn, out_type=out_type, mesh=_mesh(), scratch_types=scratch, compiler_params=_SC_PARAMS)
    return k(ef, cm, ri1, zh, zc)


def _sc_gin(xw, ri1, ci1, zh):
    out_type = jax.ShapeDtypeStruct((NSC, NP, H), jnp.float32)
    scratch = [pltpu.VMEM((WIN,), jnp.int32),
               pltpu.VMEM((WIN,), jnp.int32),
               pltpu.VMEM((WIN, H), jnp.float32),
               pltpu.VMEM_SHARED((NP, H), jnp.float32)]

    def body_fn(xw_hbm, ri_hbm, ci_hbm, zh_hbm, oa_hbm, ri_v, ci_v, buf_v, acc_s):
        cid = lax.axis_index("c")
        sid = lax.axis_index("s")
        r0 = sid * ROWS_SUB
        pltpu.sync_copy(zh_hbm.at[pl.ds(r0, ROWS_SUB)], acc_s.at[pl.ds(r0, ROWS_SUB)])
        plsc.subcore_barrier()
        wid = sid * NSC + cid

        @pl.loop(0, WPW)
        def _(j):
            base = (wid * WPW + j) * WIN
            pltpu.sync_copy(ri_hbm.at[pl.ds(base, WIN)], ri_v)
            pltpu.sync_copy(ci_hbm.at[pl.ds(base, WIN)], ci_v)
            pltpu.sync_copy(xw_hbm.at[ri_v], buf_v)
            pltpu.sync_copy(buf_v, acc_s.at[ci_v], add=True)

        plsc.subcore_barrier()
        pltpu.sync_copy(acc_s.at[pl.ds(r0, ROWS_SUB)], oa_hbm.at[cid, pl.ds(r0, ROWS_SUB)])

    k = pl.kernel(body_fn, out_type=out_type, mesh=_mesh(), scratch_types=scratch, compiler_params=_SC_PARAMS)
    return k(xw, ri1, ci1, zh)


def _full(shape):
    nd = len(shape)
    return pl.BlockSpec(shape, lambda i: (0,) * nd)


def _tc_prelude(hp, w1a, w1b):
    def kfn(h_ref, wa_ref, wb_ref, a_ref, b_ref):
        hb = h_ref[...]
        a_ref[...] = jnp.dot(hb, wa_ref[...], preferred_element_type=jnp.float32).astype(jnp.float32)
        b_ref[...] = jnp.dot(hb, wb_ref[...], preferred_element_type=jnp.float32).astype(jnp.float32)

    return pl.pallas_call(
        kfn,
        grid=(NP // BN,),
        in_specs=[pl.BlockSpec((BN, H), lambda i: (i, 0)), _full((H, H)), _full((H, H))],
        out_specs=[pl.BlockSpec((BN, H), lambda i: (i, 0)),
                   pl.BlockSpec((BN, H), lambda i: (i, 0))],
        out_shape=(jax.ShapeDtypeStruct((NP, H), jnp.float32),
                   jax.ShapeDtypeStruct((NP, H), jnp.float32)),
    )(hp, w1a, w1b)


def _tc_edge(ga, gb, gcr, gcc, w1r, eb1, ew2, eb2, cw1, cb1, cw2t, ne):
    def kfn(ga_ref, gb_ref, gcr_ref, gcc_ref, w1r_ref, eb1_ref, ew2_ref, eb2_ref,
            cw1_ref, cb1_ref, cw2t_ref, ef_ref, cm_ref):
        cd = gcr_ref[...] - gcc_ref[...]
        radial = jnp.sum(cd * cd, axis=1, keepdims=True)
        pre = (ga_ref[...].astype(jnp.float32) + gb_ref[...].astype(jnp.float32)
               + radial * w1r_ref[...] + eb1_ref[...])
        ef1 = _silu(pre)
        ef2 = _silu(jnp.dot(ef1, ew2_ref[...], preferred_element_type=jnp.float32) + eb2_ref[...])
        u = _silu(jnp.dot(ef2, cw1_ref[...], preferred_element_type=jnp.float32) + cb1_ref[...])
        m = jnp.sum(u * cw2t_ref[...], axis=1, keepdims=True)
        lane = lax.broadcasted_iota(jnp.int32, (BW, CW), 1)
        cm = jnp.where(lane == 3, 1.0, cd * m)
        ef_ref[...] = ef2
        cm_ref[...] = cm

    return pl.pallas_call(
        kfn,
        grid=(ne // BW,),
        in_specs=[pl.BlockSpec((BW, H), lambda i: (i, 0)),
                  pl.BlockSpec((BW, H), lambda i: (i, 0)),
                  pl.BlockSpec((BW, CW), lambda i: (i, 0)),
                  pl.BlockSpec((BW, CW), lambda i: (i, 0)),
                  _full((1, H)), _full((1, H)), _full((H, H)), _full((1, H)),
                  _full((H, H)), _full((1, H)), _full((1, H))],
        out_specs=[pl.BlockSpec((BW, H), lambda i: (i, 0)),
                   pl.BlockSpec((BW, CW), lambda i: (i, 0))],
        out_shape=(jax.ShapeDtypeStruct((ne, H), jnp.float32),
                   jax.ShapeDtypeStruct((ne, CW), jnp.float32)),
    )(ga, gb, gcr, gcc, w1r, eb1, ew2, eb2, cw1, cb1, cw2t)


def _tc_node0(hp, aggp, aggp2, caccp, caccp2, coordp, nw1a, nw1b, nb1, nw2, nb2, w1a_n, w1b_n):
    def kfn(h_ref, ap_ref, ap2_ref, cp_ref, cp2_ref, co_ref, nw1a_ref, nw1b_ref, nb1_ref, nw2_ref,
            nb2_ref, wa_ref, wb_ref, hn_ref, con_ref, an_ref, bn_ref):
        agg = ap_ref[0] + ap_ref[1] + ap2_ref[0] + ap2_ref[1]
        o1 = _silu(jnp.dot(h_ref[...], nw1a_ref[...], preferred_element_type=jnp.float32)
                   + jnp.dot(agg, nw1b_ref[...], preferred_element_type=jnp.float32)
                   + nb1_ref[...])
        hn = jnp.dot(o1, nw2_ref[...], preferred_element_type=jnp.float32) + nb2_ref[...]
        cacc = cp_ref[0] + cp_ref[1] + cp2_ref[0] + cp2_ref[1]
        cnt = jnp.clip(cacc[:, 3:4], 1.0, None)
        lane = lax.broadcasted_iota(jnp.int32, (BN, CW), 1)
        con = co_ref[...] + jnp.where(lane < 3, cacc / cnt, 0.0)
        hn_ref[...] = hn
        con_ref[...] = con
        an_ref[...] = jnp.dot(hn, wa_ref[...], preferred_element_type=jnp.float32).astype(jnp.float32)
        bn_ref[...] = jnp.dot(hn, wb_ref[...], preferred_element_type=jnp.float32).astype(jnp.float32)

    return pl.pallas_call(
        kfn,
        grid=(NP // BN,),
        in_specs=[pl.BlockSpec((BN, H), lambda i: (i, 0)),
                  pl.BlockSpec((NSC, BN, H), lambda i: (0, i, 0)),
                  pl.BlockSpec((NSC, BN, H), lambda i: (0, i, 0)),
                  pl.BlockSpec((NSC, BN, CW), lambda i: (0, i, 0)),
                  pl.BlockSpec((NSC, BN, CW), lambda i: (0, i, 0)),
                  pl.BlockSpec((BN, CW), lambda i: (i, 0)),
                  _full((H, H)), _full((H, H)), _full((1, H)), _full((H, H)),
                  _full((1, H)), _full((H, H)), _full((H, H))],
        out_specs=[pl.BlockSpec((BN, H), lambda i: (i, 0)),
                   pl.BlockSpec((BN, CW), lambda i: (i, 0)),
                   pl.BlockSpec((BN, H), lambda i: (i, 0)),
                   pl.BlockSpec((BN, H), lambda i: (i, 0))],
        out_shape=(jax.ShapeDtypeStruct((NP, H), jnp.float32),
                   jax.ShapeDtypeStruct((NP, CW), jnp.float32),
                   jax.ShapeDtypeStruct((NP, H), jnp.float32),
                   jax.ShapeDtypeStruct((NP, H), jnp.float32)),
    )(hp, aggp, aggp2, caccp, caccp2, coordp, nw1a, nw1b, nb1, nw2, nb2, w1a_n, w1b_n)


def _tc_node1(hp, aggp, aggp2, caccp, caccp2, coordp, coord0p, nw1a, nw1b, nb1, nw2, nb2, gw1a, gw1b):
    def kfn(h_ref, ap_ref, ap2_ref, cp_ref, cp2_ref, co_ref, co0_ref, nw1a_ref, nw1b_ref, nb1_ref,
            nw2_ref, nb2_ref, ga_ref, gb_ref, xw_ref):
        agg = ap_ref[0] + ap_ref[1] + ap2_ref[0] + ap2_ref[1]
        o1 = _silu(jnp.dot(h_ref[...], nw1a_ref[...], preferred_element_type=jnp.float32)
                   + jnp.dot(agg, nw1b_ref[...], preferred_element_type=jnp.float32)
                   + nb1_ref[...])
        hn = h_ref[...] + jnp.dot(o1, nw2_ref[...], preferred_element_type=jnp.float32) + nb2_ref[...]
        cacc = cp_ref[0] + cp_ref[1] + cp2_ref[0] + cp2_ref[1]
        cnt = jnp.clip(cacc[:, 3:4], 1.0, None)
        lane = lax.broadcasted_iota(jnp.int32, (BN, CW), 1)
        con = co_ref[...] + jnp.where(lane < 3, cacc / cnt, 0.0)
        dd = con - co0_ref[...]
        delta = jnp.sqrt(jnp.sum(dd * dd, axis=1, keepdims=True))
        xw_ref[...] = (jnp.dot(hn, ga_ref[...], preferred_element_type=jnp.float32)
                       + delta * gb_ref[...])

    return pl.pallas_call(
        kfn,
        grid=(NP // BN,),
        in_specs=[pl.BlockSpec((BN, H), lambda i: (i, 0)),
                  pl.BlockSpec((NSC, BN, H), lambda i: (0, i, 0)),
                  pl.BlockSpec((NSC, BN, H), lambda i: (0, i, 0)),
                  pl.BlockSpec((NSC, BN, CW), lambda i: (0, i, 0)),
                  pl.BlockSpec((NSC, BN, CW), lambda i: (0, i, 0)),
                  pl.BlockSpec((BN, CW), lambda i: (i, 0)),
                  pl.BlockSpec((BN, CW), lambda i: (i, 0)),
                  _full((H, H)), _full((H, H)), _full((1, H)), _full((H, H)),
                  _full((1, H)), _full((H, H)), _full((1, H))],
        out_specs=[pl.BlockSpec((BN, H), lambda i: (i, 0))],
        out_shape=(jax.ShapeDtypeStruct((NP, H), jnp.float32),),
    )(hp, aggp, aggp2, caccp, caccp2, coordp, coord0p, nw1a, nw1b, nb1, nw2, nb2, gw1a, gw1b)


def _tc_final(xw, gp, batc, g_b1, g_gamma, g_beta, g_w2, g_b2, g0,
              cw1m, cw1x, cw1g, c_b1, c_gamma, c_beta, c_w2, c_b2):
    nblk = NP // BN
    binv = float((1.0 + 1e-5) ** -0.5)

    def kfn(xw_ref, gp_ref, bat_ref, gb1_ref, gg_ref, gbe_ref, gw2_ref, gb2_ref,
            g0_ref, c1m_ref, c1x_ref, c1g_ref, cb1_ref, cg_ref, cbe_ref,
            cw2_ref, cb2_ref, out_ref, mean_s, max_s, cnt_s):
        step = pl.program_id(0)

        @pl.when(step == 0)
        def _():
            mean_s[...] = jnp.zeros((NG, H), jnp.float32)
            cnt_s[...] = jnp.zeros((NG, H), jnp.float32)
            max_s[...] = jnp.full((NG, H), -jnp.inf, jnp.float32)

        y = xw_ref[...] + gp_ref[0] + gp_ref[1] + gb1_ref[...]
        y = y * binv * gg_ref[...] + gbe_ref[...]
        y = jnp.maximum(y, 0.0)
        hf = _elu01(jnp.dot(y, gw2_ref[...], preferred_element_type=jnp.float32) + gb2_ref[...])

        bat = bat_ref[...]
        grp = lax.broadcasted_iota(jnp.int32, (NG, BN), 0)
        pmat = (bat.reshape(1, BN) == grp).astype(jnp.float32)
        mean_s[...] += jnp.dot(pmat, hf, preferred_element_type=jnp.float32)
        cnt_s[...] += jnp.broadcast_to(jnp.sum(pmat, axis=1, keepdims=True), (NG, H))
        for g in range(NG):
            mg = jnp.where(bat == g, hf, -jnp.inf)
            red = jnp.max(mg, axis=0, keepdims=True)
            max_s[g:g + 1, :] = jnp.maximum(max_s[g:g + 1, :], red)

        @pl.when(step == nblk - 1)
        def _():
            cnt = jnp.clip(cnt_s[...], 1.0, None)
            meanp = mean_s[...] / cnt
            maxp = max_s[...]
            z = (jnp.dot(meanp, c1m_ref[...], preferred_element_type=jnp.float32)
                 + jnp.dot(maxp, c1x_ref[...], preferred_element_type=jnp.float32)
                 + jnp.dot(g0_ref[...], c1g_ref[...], preferred_element_type=jnp.float32)
                 + cb1_ref[...])
            z = _elu01(z)
            z = z * binv * cg_ref[...] + cbe_ref[...]
            z = jnp.dot(z, cw2_ref[...], preferred_element_type=jnp.float32) + cb2_ref[...]
            z = z - jnp.max(z, axis=1, keepdims=True)
            ez = jnp.exp(z)
            out_ref[...] = ez / jnp.sum(ez, axis=1, keepdims=True)

    return pl.pallas_call(
        kfn,
        grid=(nblk,),
        in_specs=[pl.BlockSpec((BN, H), lambda i: (i, 0)),
                  pl.BlockSpec((NSC, BN, H), lambda i: (0, i, 0)),
                  pl.BlockSpec((BN, 1), lambda i: (i, 0)),
                  _full((1, H)), _full((1, H)), _full((1, H)), _full((H, H)),
                  _full((1, H)), _full((NG, NG)), _full((H, H)), _full((H, H)),
                  _full((NG, H)), _full((1, H)), _full((1, H)), _full((1, H)),
                  _full((H, 4)), _full((1, 4))],
        out_specs=pl.BlockSpec((NG, 4), lambda i: (0, 0)),
        out_shape=jax.ShapeDtypeStruct((NG, 4), jnp.float32),
        scratch_shapes=[pltpu.VMEM((NG, H), jnp.float32),
                        pltpu.VMEM((NG, H), jnp.float32),
                        pltpu.VMEM((NG, H), jnp.float32)],
    )(xw, gp, batc, g_b1, g_gamma, g_beta, g_w2, g_b2, g0,
      cw1m, cw1x, cw1g, c_b1, c_gamma, c_beta, c_w2, c_b2)


def kernel(h0, coord0, g0, edge_index, batch, e0_ew1, e0_eb1, e0_ew2, e0_eb2, e0_nw1, e0_nb1, e0_nw2, e0_nb2, e0_cw1, e0_cb1, e0_cw2, e1_ew1, e1_eb1, e1_ew2, e1_eb2, e1_nw1, e1_nb1, e1_nw2, e1_nb2, e1_cw1, e1_cb1, e1_cw2, g_w1, g_b1, g_gamma, g_beta, g_w2, g_b2, c_w1, c_b1, c_gamma, c_beta, c_w2, c_b2):
    f32 = jnp.float32
    hp = jnp.zeros((NP, H), f32).at[:NN].set(h0)
    coord0p = jnp.zeros((NP, CW), f32).at[:NN, :3].set(coord0)
    row = jnp.full((EP,), NN, jnp.int32).at[:EE].set(edge_index[0])
    col = jnp.full((EP,), NN, jnp.int32).at[:EE].set(edge_index[1])
    batc = jnp.full((NP, 1), NG, jnp.int32).at[:NN, 0].set(batch)
    zh = jnp.zeros((NP, H), f32)
    zc = jnp.zeros((NP, CW), f32)

    def rowvec(v):
        return v.reshape(1, -1)

    coordp = coord0p
    e_params = [
        (e0_ew1, e0_eb1, e0_ew2, e0_eb2, e0_nw1, e0_nb1, e0_nw2, e0_nb2, e0_cw1, e0_cb1, e0_cw2),
        (e1_ew1, e1_eb1, e1_ew2, e1_eb2, e1_nw1, e1_nb1, e1_nw2, e1_nb2, e1_cw1, e1_cb1, e1_cw2),
    ]

    NE2 = EP // 2
    rows_c = (row[:NE2], row[NE2:])
    cols_c = (col[:NE2], col[NE2:])

    def layer_partials(a_t, b_t, coordp, ew1, eb1, ew2, eb2, cw1, cb1, cw2):
        parts = []
        for c in range(2):
            ga, gb, gcr, gcc = _sc_gather(a_t, b_t, coordp, rows_c[c], cols_c[c], NE2)
            ef, cm = _tc_edge(ga, gb, gcr, gcc, rowvec(ew1[2 * H]), rowvec(eb1),
                              ew2, rowvec(eb2), cw1, rowvec(cb1),
                              cw2.reshape(1, H), NE2)
            parts.append(_sc_scatter(ef, cm, rows_c[c], zh, zc, NE2))
        return parts[0][0], parts[1][0], parts[0][1], parts[1][1]

    (ew1, eb1, ew2, eb2, nw1, nb1, nw2, nb2, cw1, cb1, cw2) = e_params[0]
    a_t, b_t = _tc_prelude(hp, ew1[0:H], ew1[H:2 * H])
    aggp, aggp2, caccp, caccp2 = layer_partials(a_t, b_t, coordp, ew1, eb1, ew2,
                                                eb2, cw1, cb1, cw2)
    ew1n = e_params[1][0]
    hp, coordp, a_t, b_t = _tc_node0(hp, aggp, aggp2, caccp, caccp2, coordp,
                                     nw1[0:H], nw1[H:2 * H], rowvec(nb1),
                                     nw2, rowvec(nb2), ew1n[0:H], ew1n[H:2 * H])

    (ew1, eb1, ew2, eb2, nw1, nb1, nw2, nb2, cw1, cb1, cw2) = e_params[1]
    aggp, aggp2, caccp, caccp2 = layer_partials(a_t, b_t, coordp, ew1, eb1, ew2,
                                                eb2, cw1, cb1, cw2)
    xw = _tc_node1(hp, aggp, aggp2, caccp, caccp2, coordp, coord0p,
                   nw1[0:H], nw1[H:2 * H], rowvec(nb1), nw2, rowvec(nb2),
                   g_w1[0:H], rowvec(g_w1[H]))[0]

    gp = _sc_gin(xw, row, col, zh)

    return _tc_final(xw, gp, batc, rowvec(g_b1), rowvec(g_gamma), rowvec(g_beta),
                     g_w2, rowvec(g_b2), g0, c_w1[0:H], c_w1[H:2 * H],
                     c_w1[2 * H:2 * H + NG], rowvec(c_b1), rowvec(c_gamma),
                     rowvec(c_beta), c_w2, rowvec(c_b2))


# speedup vs baseline: 1.4792x; 1.0628x over previous
import functools

import jax
import jax.numpy as jnp
from jax import lax
from jax.experimental import pallas as pl
from jax.experimental.pallas import tpu as pltpu
from jax.experimental.pallas import tpu_sc as plsc

NN = 10000
NP = 10240
EE = 320000
EP = 327680
H = 128
NG = 16
CW = 16
WIN = 128
BW = 512
BN = 512
NSC = 2
NSUB = 16
ROWS_SUB = NP // NSUB
WINS = EP // WIN
NWORK = NSC * NSUB
WPW = WINS // NWORK


def _silu(x):
    return x * jax.nn.sigmoid(x)


def _elu01(x):
    return jnp.where(x > 0, x, 0.1 * (jnp.exp(x) - 1.0))


def _mesh():
    return plsc.VectorSubcoreMesh(core_axis_name="c", subcore_axis_name="s")


_SC_PARAMS = pltpu.CompilerParams(use_tc_tiling_on_sc=False)


def _sc_gather(a_t, b_t, c_t, ri1, ci1, ne):
    wpw = ne // WIN // NWORK
    out_type = (jax.ShapeDtypeStruct((ne, H), jnp.float32),
                jax.ShapeDtypeStruct((ne, H), jnp.float32),
                jax.ShapeDtypeStruct((ne, CW), jnp.float32),
                jax.ShapeDtypeStruct((ne, CW), jnp.float32))
    scratch = [pltpu.VMEM((wpw * WIN,), jnp.int32),
               pltpu.VMEM((wpw * WIN,), jnp.int32),
               pltpu.VMEM((WIN, H), jnp.float32),
               pltpu.VMEM((WIN, H), jnp.float32),
               pltpu.VMEM((WIN, H), jnp.float32),
               pltpu.VMEM((WIN, H), jnp.float32),
               pltpu.VMEM((WIN, CW), jnp.float32),
               pltpu.VMEM((WIN, CW), jnp.float32),
               pltpu.VMEM((WIN, CW), jnp.float32),
               pltpu.VMEM((WIN, CW), jnp.float32),
               pltpu.SemaphoreType.DMA,
               pltpu.SemaphoreType.DMA]

    def body_fn(a_hbm, b_hbm, c_hbm, ri_hbm, ci_hbm,
                ga_hbm, gb_hbm, gcr_hbm, gcc_hbm,
                ri_a, ci_a, ga0, ga1, gb0, gb1, cr0, cr1, cc0, cc1, sem0, sem1):
        cid = lax.axis_index("c")
        sid = lax.axis_index("s")
        wid = sid * NSC + cid
        ebase = wid * wpw * WIN
        pltpu.sync_copy(ri_hbm.at[pl.ds(ebase, wpw * WIN)], ri_a)
        pltpu.sync_copy(ci_hbm.at[pl.ds(ebase, wpw * WIN)], ci_a)

        def fire(j, ga, gb, cr, cc, sem):
            rs = ri_a.at[pl.ds(j * WIN, WIN)]
            cs = ci_a.at[pl.ds(j * WIN, WIN)]
            pltpu.async_copy(a_hbm.at[rs], ga, sem)
            pltpu.async_copy(b_hbm.at[cs], gb, sem)
            pltpu.async_copy(c_hbm.at[rs], cr, sem)
            pltpu.async_copy(c_hbm.at[cs], cc, sem)

        def drain(ga, gb, cr, cc, sem):
            pltpu.make_async_copy(a_hbm.at[pl.ds(0, WIN)], ga, sem).wait()
            pltpu.make_async_copy(b_hbm.at[pl.ds(0, WIN)], gb, sem).wait()
            pltpu.make_async_copy(c_hbm.at[pl.ds(0, WIN)], cr, sem).wait()
            pltpu.make_async_copy(c_hbm.at[pl.ds(0, WIN)], cc, sem).wait()

        def store(j, ga, gb, cr, cc):
            base = ebase + j * WIN
            pltpu.sync_copy(ga, ga_hbm.at[pl.ds(base, WIN)])
            pltpu.sync_copy(gb, gb_hbm.at[pl.ds(base, WIN)])
            pltpu.sync_copy(cr, gcr_hbm.at[pl.ds(base, WIN)])
            pltpu.sync_copy(cc, gcc_hbm.at[pl.ds(base, WIN)])

        fire(0, ga0, gb0, cr0, cc0, sem0)

        @pl.loop(0, wpw // 2)
        def _(k):
            j0 = 2 * k
            drain(ga0, gb0, cr0, cc0, sem0)
            fire(j0 + 1, ga1, gb1, cr1, cc1, sem1)
            store(j0, ga0, gb0, cr0, cc0)
            drain(ga1, gb1, cr1, cc1, sem1)

            @pl.when(k < wpw // 2 - 1)
            def _():
                fire(j0 + 2, ga0, gb0, cr0, cc0, sem0)

            store(j0 + 1, ga1, gb1, cr1, cc1)

    k = pl.kernel(body_fn, out_type=out_type, mesh=_mesh(), scratch_types=scratch,
                  compiler_params=_SC_PARAMS)
    return k(a_t, b_t, c_t, ri1, ci1)


def _sc_scatter(ef, cm, ri1, zh, zc, ne):
    wpw = ne // WIN // NWORK
    out_type = (jax.ShapeDtypeStruct((NSC, NP, H), jnp.float32),
                jax.ShapeDtypeStruct((NSC, NP, CW), jnp.float32))
    scratch = [pltpu.VMEM((WIN,), jnp.int32),
               pltpu.VMEM((WIN,), jnp.int32),
               pltpu.VMEM((WIN, H), jnp.float32),
               pltpu.VMEM((WIN, H), jnp.float32),
               pltpu.VMEM((WIN, CW), jnp.float32),
               pltpu.VMEM((WIN, CW), jnp.float32),
               pltpu.VMEM_SHARED((NP, H), jnp.float32),
               pltpu.VMEM_SHARED((NP, CW), jnp.float32),
               pltpu.SemaphoreType.DMA,
               pltpu.SemaphoreType.DMA]

    def body_fn(ef_hbm, cm_hbm, ri_hbm, zh_hbm, zc_hbm, oa_hbm, oc_hbm,
                ix0, ix1, ef0, ef1, cm0, cm1, acc_s, cacc_s, sem0, sem1):
        cid = lax.axis_index("c")
        sid = lax.axis_index("s")
        r0 = sid * ROWS_SUB
        pltpu.sync_copy(zh_hbm.at[pl.ds(r0, ROWS_SUB)], acc_s.at[pl.ds(r0, ROWS_SUB)])
        pltpu.sync_copy(zc_hbm.at[pl.ds(r0, ROWS_SUB)], cacc_s.at[pl.ds(r0, ROWS_SUB)])
        wid = sid * NSC + cid
        ebase = wid * wpw * WIN
        plsc.subcore_barrier()

        def fire(j, ixb, efb, cmb, sem):
            pltpu.async_copy(ri_hbm.at[pl.ds(ebase + j * WIN, WIN)], ixb, sem)
            pltpu.async_copy(ef_hbm.at[pl.ds(ebase + j * WIN, WIN)], efb, sem)
            pltpu.async_copy(cm_hbm.at[pl.ds(ebase + j * WIN, WIN)], cmb, sem)

        def drain(ixb, efb, cmb, sem):
            pltpu.make_async_copy(ri_hbm.at[pl.ds(0, WIN)], ixb, sem).wait()
            pltpu.make_async_copy(ef_hbm.at[pl.ds(0, WIN)], efb, sem).wait()
            pltpu.make_async_copy(cm_hbm.at[pl.ds(0, WIN)], cmb, sem).wait()

        def scat(ixb, efb, cmb):
            pltpu.sync_copy(efb, acc_s.at[ixb], add=True)
            pltpu.sync_copy(cmb, cacc_s.at[ixb], add=True)

        fire(0, ix0, ef0, cm0, sem0)

        @pl.loop(0, wpw // 2)
        def _(k):
            j0 = 2 * k
            drain(ix0, ef0, cm0, sem0)
            fire(j0 + 1, ix1, ef1, cm1, sem1)
            scat(ix0, ef0, cm0)
            drain(ix1, ef1, cm1, sem1)

            @pl.when(k < wpw // 2 - 1)
            def _():
                fire(j0 + 2, ix0, ef0, cm0, sem0)

            scat(ix1, ef1, cm1)

        plsc.subcore_barrier()
        pltpu.sync_copy(acc_s.at[pl.ds(r0, ROWS_SUB)], oa_hbm.at[cid, pl.ds(r0, ROWS_SUB)])
        pltpu.sync_copy(cacc_s.at[pl.ds(r0, ROWS_SUB)], oc_hbm.at[cid, pl.ds(r0, ROWS_SUB)])

    k = pl.kernel(body_fn, out_type=out_type, mesh=_mesh(), scratch_types=scratch,
                  compiler_params=_SC_PARAMS)
    return k(ef, cm, ri1, zh, zc)


def _sc_gin(xw, ri1, ci1, zh):
    wpw = EP // WIN // NWORK
    out_type = jax.ShapeDtypeStruct((NSC, NP, H), jnp.float32)
    scratch = [pltpu.VMEM((wpw * WIN,), jnp.int32),
               pltpu.VMEM((WIN,), jnp.int32),
               pltpu.VMEM((WIN,), jnp.int32),
               pltpu.VMEM((WIN, H), jnp.float32),
               pltpu.VMEM((WIN, H), jnp.float32),
               pltpu.VMEM_SHARED((NP, H), jnp.float32),
               pltpu.SemaphoreType.DMA,
               pltpu.SemaphoreType.DMA]

    def body_fn(xw_hbm, ri_hbm, ci_hbm, zh_hbm, oa_hbm,
                ri_a, cx0, cx1, b0, b1, acc_s, sem0, sem1):
        cid = lax.axis_index("c")
        sid = lax.axis_index("s")
        r0 = sid * ROWS_SUB
        pltpu.sync_copy(zh_hbm.at[pl.ds(r0, ROWS_SUB)], acc_s.at[pl.ds(r0, ROWS_SUB)])
        wid = sid * NSC + cid
        ebase = wid * wpw * WIN
        pltpu.sync_copy(ri_hbm.at[pl.ds(ebase, wpw * WIN)], ri_a)
        plsc.subcore_barrier()

        def fire(j, cxb, b, sem):
            pltpu.async_copy(ci_hbm.at[pl.ds(ebase + j * WIN, WIN)], cxb, sem)
            pltpu.async_copy(xw_hbm.at[ri_a.at[pl.ds(j * WIN, WIN)]], b, sem)

        def drain(cxb, b, sem):
            pltpu.make_async_copy(ci_hbm.at[pl.ds(0, WIN)], cxb, sem).wait()
            pltpu.make_async_copy(xw_hbm.at[pl.ds(0, WIN)], b, sem).wait()

        def scat(cxb, b):
            pltpu.sync_copy(b, acc_s.at[cxb], add=True)

        fire(0, cx0, b0, sem0)

        @pl.loop(0, wpw // 2)
        def _(k):
            j0 = 2 * k
            drain(cx0, b0, sem0)
            fire(j0 + 1, cx1, b1, sem1)
            scat(cx0, b0)
            drain(cx1, b1, sem1)

            @pl.when(k < wpw // 2 - 1)
            def _():
                fire(j0 + 2, cx0, b0, sem0)

            scat(cx1, b1)

        plsc.subcore_barrier()
        pltpu.sync_copy(acc_s.at[pl.ds(r0, ROWS_SUB)], oa_hbm.at[cid, pl.ds(r0, ROWS_SUB)])

    k = pl.kernel(body_fn, out_type=out_type, mesh=_mesh(), scratch_types=scratch,
                  compiler_params=_SC_PARAMS)
    return k(xw, ri1, ci1, zh)


def _full(shape):
    nd = len(shape)
    return pl.BlockSpec(shape, lambda i: (0,) * nd)


def _tc_prelude(hp, w1a, w1b):
    def kfn(h_ref, wa_ref, wb_ref, a_ref, b_ref):
        hb = h_ref[...]
        a_ref[...] = jnp.dot(hb, wa_ref[...], preferred_element_type=jnp.float32).astype(jnp.float32)
        b_ref[...] = jnp.dot(hb, wb_ref[...], preferred_element_type=jnp.float32).astype(jnp.float32)

    return pl.pallas_call(
        kfn,
        grid=(NP // BN,),
        in_specs=[pl.BlockSpec((BN, H), lambda i: (i, 0)), _full((H, H)), _full((H, H))],
        out_specs=[pl.BlockSpec((BN, H), lambda i: (i, 0)),
                   pl.BlockSpec((BN, H), lambda i: (i, 0))],
        out_shape=(jax.ShapeDtypeStruct((NP, H), jnp.float32),
                   jax.ShapeDtypeStruct((NP, H), jnp.float32)),
    )(hp, w1a, w1b)


def _tc_edge(ga, gb, gcr, gcc, w1r, eb1, ew2, eb2, cw1, cb1, cw2t, ne):
    def kfn(ga_ref, gb_ref, gcr_ref, gcc_ref, w1r_ref, eb1_ref, ew2_ref, eb2_ref,
            cw1_ref, cb1_ref, cw2t_ref, ef_ref, cm_ref):
        cd = gcr_ref[...] - gcc_ref[...]
        radial = jnp.sum(cd * cd, axis=1, keepdims=True)
        pre = (ga_ref[...].astype(jnp.float32) + gb_ref[...].astype(jnp.float32)
               + radial * w1r_ref[...] + eb1_ref[...])
        ef1 = _silu(pre)
        ef2 = _silu(jnp.dot(ef1, ew2_ref[...], preferred_element_type=jnp.float32) + eb2_ref[...])
        u = _silu(jnp.dot(ef2, cw1_ref[...], preferred_element_type=jnp.float32) + cb1_ref[...])
        m = jnp.sum(u * cw2t_ref[...], axis=1, keepdims=True)
        lane = lax.broadcasted_iota(jnp.int32, (BW, CW), 1)
        cm = jnp.where(lane == 3, 1.0, cd * m)
        ef_ref[...] = ef2
        cm_ref[...] = cm

    return pl.pallas_call(
        kfn,
        grid=(ne // BW,),
        in_specs=[pl.BlockSpec((BW, H), lambda i: (i, 0)),
                  pl.BlockSpec((BW, H), lambda i: (i, 0)),
                  pl.BlockSpec((BW, CW), lambda i: (i, 0)),
                  pl.BlockSpec((BW, CW), lambda i: (i, 0)),
                  _full((1, H)), _full((1, H)), _full((H, H)), _full((1, H)),
                  _full((H, H)), _full((1, H)), _full((1, H))],
        out_specs=[pl.BlockSpec((BW, H), lambda i: (i, 0)),
                   pl.BlockSpec((BW, CW), lambda i: (i, 0))],
        out_shape=(jax.ShapeDtypeStruct((ne, H), jnp.float32),
                   jax.ShapeDtypeStruct((ne, CW), jnp.float32)),
    )(ga, gb, gcr, gcc, w1r, eb1, ew2, eb2, cw1, cb1, cw2t)


def _tc_node0(hp, aggp, aggp2, caccp, caccp2, coordp, nw1a, nw1b, nb1, nw2, nb2, w1a_n, w1b_n):
    def kfn(h_ref, ap_ref, ap2_ref, cp_ref, cp2_ref, co_ref, nw1a_ref, nw1b_ref, nb1_ref, nw2_ref,
            nb2_ref, wa_ref, wb_ref, hn_ref, con_ref, an_ref, bn_ref):
        agg = ap_ref[0] + ap_ref[1] + ap2_ref[0] + ap2_ref[1]
        o1 = _silu(jnp.dot(h_ref[...], nw1a_ref[...], preferred_element_type=jnp.float32)
                   + jnp.dot(agg, nw1b_ref[...], preferred_element_type=jnp.float32)
                   + nb1_ref[...])
        hn = jnp.dot(o1, nw2_ref[...], preferred_element_type=jnp.float32) + nb2_ref[...]
        cacc = cp_ref[0] + cp_ref[1] + cp2_ref[0] + cp2_ref[1]
        cnt = jnp.clip(cacc[:, 3:4], 1.0, None)
        lane = lax.broadcasted_iota(jnp.int32, (BN, CW), 1)
        con = co_ref[...] + jnp.where(lane < 3, cacc / cnt, 0.0)
        hn_ref[...] = hn
        con_ref[...] = con
        an_ref[...] = jnp.dot(hn, wa_ref[...], preferred_element_type=jnp.float32).astype(jnp.float32)
        bn_ref[...] = jnp.dot(hn, wb_ref[...], preferred_element_type=jnp.float32).astype(jnp.float32)

    return pl.pallas_call(
        kfn,
        grid=(NP // BN,),
        in_specs=[pl.BlockSpec((BN, H), lambda i: (i, 0)),
                  pl.BlockSpec((NSC, BN, H), lambda i: (0, i, 0)),
                  pl.BlockSpec((NSC, BN, H), lambda i: (0, i, 0)),
                  pl.BlockSpec((NSC, BN, CW), lambda i: (0, i, 0)),
                  pl.BlockSpec((NSC, BN, CW), lambda i: (0, i, 0)),
                  pl.BlockSpec((BN, CW), lambda i: (i, 0)),
                  _full((H, H)), _full((H, H)), _full((1, H)), _full((H, H)),
                  _full((1, H)), _full((H, H)), _full((H, H))],
        out_specs=[pl.BlockSpec((BN, H), lambda i: (i, 0)),
                   pl.BlockSpec((BN, CW), lambda i: (i, 0)),
                   pl.BlockSpec((BN, H), lambda i: (i, 0)),
                   pl.BlockSpec((BN, H), lambda i: (i, 0))],
        out_shape=(jax.ShapeDtypeStruct((NP, H), jnp.float32),
                   jax.ShapeDtypeStruct((NP, CW), jnp.float32),
                   jax.ShapeDtypeStruct((NP, H), jnp.float32),
                   jax.ShapeDtypeStruct((NP, H), jnp.float32)),
    )(hp, aggp, aggp2, caccp, caccp2, coordp, nw1a, nw1b, nb1, nw2, nb2, w1a_n, w1b_n)


def _tc_node1(hp, aggp, aggp2, caccp, caccp2, coordp, coord0p, nw1a, nw1b, nb1, nw2, nb2, gw1a, gw1b):
    def kfn(h_ref, ap_ref, ap2_ref, cp_ref, cp2_ref, co_ref, co0_ref, nw1a_ref, nw1b_ref, nb1_ref,
            nw2_ref, nb2_ref, ga_ref, gb_ref, xw_ref):
        agg = ap_ref[0] + ap_ref[1] + ap2_ref[0] + ap2_ref[1]
        o1 = _silu(jnp.dot(h_ref[...], nw1a_ref[...], preferred_element_type=jnp.float32)
                   + jnp.dot(agg, nw1b_ref[...], preferred_element_type=jnp.float32)
                   + nb1_ref[...])
        hn = h_ref[...] + jnp.dot(o1, nw2_ref[...], preferred_element_type=jnp.float32) + nb2_ref[...]
        cacc = cp_ref[0] + cp_ref[1] + cp2_ref[0] + cp2_ref[1]
        cnt = jnp.clip(cacc[:, 3:4], 1.0, None)
        lane = lax.broadcasted_iota(jnp.int32, (BN, CW), 1)
        con = co_ref[...] + jnp.where(lane < 3, cacc / cnt, 0.0)
        dd = con - co0_ref[...]
        delta = jnp.sqrt(jnp.sum(dd * dd, axis=1, keepdims=True))
        xw_ref[...] = (jnp.dot(hn, ga_ref[...], preferred_element_type=jnp.float32)
                       + delta * gb_ref[...])

    return pl.pallas_call(
        kfn,
        grid=(NP // BN,),
        in_specs=[pl.BlockSpec((BN, H), lambda i: (i, 0)),
                  pl.BlockSpec((NSC, BN, H), lambda i: (0, i, 0)),
                  pl.BlockSpec((NSC, BN, H), lambda i: (0, i, 0)),
                  pl.BlockSpec((NSC, BN, CW), lambda i: (0, i, 0)),
                  pl.BlockSpec((NSC, BN, CW), lambda i: (0, i, 0)),
                  pl.BlockSpec((BN, CW), lambda i: (i, 0)),
                  pl.BlockSpec((BN, CW), lambda i: (i, 0)),
                  _full((H, H)), _full((H, H)), _full((1, H)), _full((H, H)),
                  _full((1, H)), _full((H, H)), _full((1, H))],
        out_specs=[pl.BlockSpec((BN, H), lambda i: (i, 0))],
        out_shape=(jax.ShapeDtypeStruct((NP, H), jnp.float32),),
    )(hp, aggp, aggp2, caccp, caccp2, coordp, coord0p, nw1a, nw1b, nb1, nw2, nb2, gw1a, gw1b)


def _tc_final(xw, gp, batc, g_b1, g_gamma, g_beta, g_w2, g_b2, g0,
              cw1m, cw1x, cw1g, c_b1, c_gamma, c_beta, c_w2, c_b2):
    nblk = NP // BN
    binv = float((1.0 + 1e-5) ** -0.5)

    def kfn(xw_ref, gp_ref, bat_ref, gb1_ref, gg_ref, gbe_ref, gw2_ref, gb2_ref,
            g0_ref, c1m_ref, c1x_ref, c1g_ref, cb1_ref, cg_ref, cbe_ref,
            cw2_ref, cb2_ref, out_ref, mean_s, max_s, cnt_s):
        step = pl.program_id(0)

        @pl.when(step == 0)
        def _():
            mean_s[...] = jnp.zeros((NG, H), jnp.float32)
            cnt_s[...] = jnp.zeros((NG, H), jnp.float32)
            max_s[...] = jnp.full((NG, H), -jnp.inf, jnp.float32)

        y = xw_ref[...] + gp_ref[0] + gp_ref[1] + gb1_ref[...]
        y = y * binv * gg_ref[...] + gbe_ref[...]
        y = jnp.maximum(y, 0.0)
        hf = _elu01(jnp.dot(y, gw2_ref[...], preferred_element_type=jnp.float32) + gb2_ref[...])

        bat = bat_ref[...]
        grp = lax.broadcasted_iota(jnp.int32, (NG, BN), 0)
        pmat = (bat.reshape(1, BN) == grp).astype(jnp.float32)
        mean_s[...] += jnp.dot(pmat, hf, preferred_element_type=jnp.float32)
        cnt_s[...] += jnp.broadcast_to(jnp.sum(pmat, axis=1, keepdims=True), (NG, H))
        for g in range(NG):
            mg = jnp.where(bat == g, hf, -jnp.inf)
            red = jnp.max(mg, axis=0, keepdims=True)
            max_s[g:g + 1, :] = jnp.maximum(max_s[g:g + 1, :], red)

        @pl.when(step == nblk - 1)
        def _():
            cnt = jnp.clip(cnt_s[...], 1.0, None)
            meanp = mean_s[...] / cnt
            maxp = max_s[...]
            z = (jnp.dot(meanp, c1m_ref[...], preferred_element_type=jnp.float32)
                 + jnp.dot(maxp, c1x_ref[...], preferred_element_type=jnp.float32)
                 + jnp.dot(g0_ref[...], c1g_ref[...], preferred_element_type=jnp.float32)
                 + cb1_ref[...])
            z = _elu01(z)
            z = z * binv * cg_ref[...] + cbe_ref[...]
            z = jnp.dot(z, cw2_ref[...], preferred_element_type=jnp.float32) + cb2_ref[...]
            z = z - jnp.max(z, axis=1, keepdims=True)
            ez = jnp.exp(z)
            out_ref[...] = ez / jnp.sum(ez, axis=1, keepdims=True)

    return pl.pallas_call(
        kfn,
        grid=(nblk,),
        in_specs=[pl.BlockSpec((BN, H), lambda i: (i, 0)),
                  pl.BlockSpec((NSC, BN, H), lambda i: (0, i, 0)),
                  pl.BlockSpec((BN, 1), lambda i: (i, 0)),
                  _full((1, H)), _full((1, H)), _full((1, H)), _full((H, H)),
                  _full((1, H)), _full((NG, NG)), _full((H, H)), _full((H, H)),
                  _full((NG, H)), _full((1, H)), _full((1, H)), _full((1, H)),
                  _full((H, 4)), _full((1, 4))],
        out_specs=pl.BlockSpec((NG, 4), lambda i: (0, 0)),
        out_shape=jax.ShapeDtypeStruct((NG, 4), jnp.float32),
        scratch_shapes=[pltpu.VMEM((NG, H), jnp.float32),
                        pltpu.VMEM((NG, H), jnp.float32),
                        pltpu.VMEM((NG, H), jnp.float32)],
    )(xw, gp, batc, g_b1, g_gamma, g_beta, g_w2, g_b2, g0,
      cw1m, cw1x, cw1g, c_b1, c_gamma, c_beta, c_w2, c_b2)


def kernel(h0, coord0, g0, edge_index, batch, e0_ew1, e0_eb1, e0_ew2, e0_eb2, e0_nw1, e0_nb1, e0_nw2, e0_nb2, e0_cw1, e0_cb1, e0_cw2, e1_ew1, e1_eb1, e1_ew2, e1_eb2, e1_nw1, e1_nb1, e1_nw2, e1_nb2, e1_cw1, e1_cb1, e1_cw2, g_w1, g_b1, g_gamma, g_beta, g_w2, g_b2, c_w1, c_b1, c_gamma, c_beta, c_w2, c_b2):
    f32 = jnp.float32
    hp = jnp.zeros((NP, H), f32).at[:NN].set(h0)
    coord0p = jnp.zeros((NP, CW), f32).at[:NN, :3].set(coord0)
    row = jnp.full((EP,), NN, jnp.int32).at[:EE].set(edge_index[0])
    col = jnp.full((EP,), NN, jnp.int32).at[:EE].set(edge_index[1])
    batc = jnp.full((NP, 1), NG, jnp.int32).at[:NN, 0].set(batch)
    zh = jnp.zeros((NP, H), f32)
    zc = jnp.zeros((NP, CW), f32)

    def rowvec(v):
        return v.reshape(1, -1)

    coordp = coord0p
    e_params = [
        (e0_ew1, e0_eb1, e0_ew2, e0_eb2, e0_nw1, e0_nb1, e0_nw2, e0_nb2, e0_cw1, e0_cb1, e0_cw2),
        (e1_ew1, e1_eb1, e1_ew2, e1_eb2, e1_nw1, e1_nb1, e1_nw2, e1_nb2, e1_cw1, e1_cb1, e1_cw2),
    ]

    NE2 = EP // 2
    rows_c = (row[:NE2], row[NE2:])
    cols_c = (col[:NE2], col[NE2:])

    def layer_partials(a_t, b_t, coordp, ew1, eb1, ew2, eb2, cw1, cb1, cw2):
        parts = []
        for c in range(2):
            ga, gb, gcr, gcc = _sc_gather(a_t, b_t, coordp, rows_c[c], cols_c[c], NE2)
            ef, cm = _tc_edge(ga, gb, gcr, gcc, rowvec(ew1[2 * H]), rowvec(eb1),
                              ew2, rowvec(eb2), cw1, rowvec(cb1),
                              cw2.reshape(1, H), NE2)
            parts.append(_sc_scatter(ef, cm, rows_c[c], zh, zc, NE2))
        return parts[0][0], parts[1][0], parts[0][1], parts[1][1]

    (ew1, eb1, ew2, eb2, nw1, nb1, nw2, nb2, cw1, cb1, cw2) = e_params[0]
    a_t, b_t = _tc_prelude(hp, ew1[0:H], ew1[H:2 * H])
    aggp, aggp2, caccp, caccp2 = layer_partials(a_t, b_t, coordp, ew1, eb1, ew2,
                                                eb2, cw1, cb1, cw2)
    ew1n = e_params[1][0]
    hp, coordp, a_t, b_t = _tc_node0(hp, aggp, aggp2, caccp, caccp2, coordp,
                                     nw1[0:H], nw1[H:2 * H], rowvec(nb1),
                                     nw2, rowvec(nb2), ew1n[0:H], ew1n[H:2 * H])

    (ew1, eb1, ew2, eb2, nw1, nb1, nw2, nb2, cw1, cb1, cw2) = e_params[1]
    aggp, aggp2, caccp, caccp2 = layer_partials(a_t, b_t, coordp, ew1, eb1, ew2,
                                                eb2, cw1, cb1, cw2)
    xw = _tc_node1(hp, aggp, aggp2, caccp, caccp2, coordp, coord0p,
                   nw1[0:H], nw1[H:2 * H], rowvec(nb1), nw2, rowvec(nb2),
                   g_w1[0:H], rowvec(g_w1[H]))[0]

    gp = _sc_gin(xw, row, col, zh)

    return _tc_final(xw, gp, batc, rowvec(g_b1), rowvec(g_gamma), rowvec(g_beta),
                     g_w2, rowvec(g_b2), g0, c_w1[0:H], c_w1[H:2 * H],
                     c_w1[2 * H:2 * H + NG], rowvec(c_b1), rowvec(c_gamma),
                     rowvec(c_beta), c_w2, rowvec(c_b2))
